# baseline (speedup 1.0000x reference)
import functools

import jax
import jax.numpy as jnp
from jax import lax
from jax.experimental import pallas as pl
from jax.experimental.pallas import tpu as pltpu
from jax.experimental.pallas import tpu_sc as plsc

NN = 10000
FD = 128
NE = 320000
NC = 2
NS = 16
NW = NC * NS
NPAD = 10240
RPT = NPAD // NS
CH = 128
RCH = RPT // CH
EC = 125
ECW = NE // NW // EC
QC = 16
NSEG = ECW // QC
EPS = 1e-5


def _mesh():
    return plsc.VectorSubcoreMesh(core_axis_name="c", subcore_axis_name="s")


def _deg_body(dst_hbm, out0_hbm, out1_hbm, dsts_v, ones_v, stg1,
              sem0, sem1, sem2, sem3, acc):
    c = lax.axis_index("c")
    s = lax.axis_index("s")
    w = c * NS + s
    slab = pl.ds(s * RPT, RPT)
    for t in range(128 // 16):
        ones_v[pl.ds(16 * t, 16)] = jnp.ones((16,), jnp.float32)
    for t in range(RPT // 16):
        stg1[pl.ds(16 * t, 16)] = jnp.zeros((16,), jnp.float32)
    pltpu.sync_copy(stg1, acc.at[slab])
    pltpu.sync_copy(dst_hbm.at[pl.ds(w * ECW, ECW), :], dsts_v)
    plsc.subcore_barrier()
    sems = [sem0, sem1, sem2, sem3]

    def body(i, carry):
        descs = []
        for b in range(4):
            descs.append(pltpu.async_copy(
                ones_v.at[pl.ds(0, EC)], acc.at[dsts_v.at[i * 4 + b]],
                sems[b], add=True))
        for d in descs:
            d.wait()
        return carry

    lax.fori_loop(0, ECW // 4, body, 0)
    plsc.subcore_barrier()
    pltpu.sync_copy(acc.at[slab], stg1)

    @pl.when(c == 0)
    def _():
        pltpu.sync_copy(stg1, out0_hbm.at[slab])

    @pl.when(c == 1)
    def _():
        pltpu.sync_copy(stg1, out1_hbm.at[slab])


def _deg_call(dstp):
    k = pl.kernel(
        _deg_body,
        out_type=(
            jax.ShapeDtypeStruct((NPAD,), jnp.float32),
            jax.ShapeDtypeStruct((NPAD,), jnp.float32),
        ),
        mesh=_mesh(),
        scratch_types=[
            pltpu.VMEM((ECW, EC), jnp.int32),
            pltpu.VMEM((128,), jnp.float32),
            pltpu.VMEM((RPT,), jnp.float32),
            pltpu.SemaphoreType.DMA,
            pltpu.SemaphoreType.DMA,
            pltpu.SemaphoreType.DMA,
            pltpu.SemaphoreType.DMA,
            pltpu.VMEM_SHARED((NPAD,), jnp.float32),
        ],
    )
    return k(dstp)


def _agg_body(hp_hbm, src_hbm, dst_hbm, out0_hbm, out1_hbm,
              src0_v, dst0_v, src1_v, dst1_v,
              ra, rb_, gsa, gsb, ssa, ssb, isem, acc):
    c = lax.axis_index("c")
    s = lax.axis_index("s")
    w = c * NS + s

    def _ld(j, buf, sem):
        return pltpu.async_copy(hp_hbm.at[pl.ds(s * RPT + j * CH, CH), :],
                                buf, sem)

    dl = {0: _ld(0, ra, gsa), 1: _ld(1, rb_, gsb)}
    for j in range(RCH):
        buf, gsem, ssem = (ra, gsa, ssa) if j % 2 == 0 else (rb_, gsb, ssb)
        dl[j].wait()
        pltpu.async_copy(buf, acc.at[pl.ds(s * RPT + j * CH, CH), :],
                         ssem).wait()
        if j + 2 < RCH:
            dl[j + 2] = _ld(j + 2, buf, gsem)
    plsc.subcore_barrier()

    ras = ra.at[pl.ds(0, EC), :]
    rbs = rb_.at[pl.ds(0, EC), :]
    idxsets = [(src0_v, dst0_v), (src1_v, dst1_v)]

    def _refill(q, p):
        sv, dv = idxsets[p]
        pltpu.async_copy(src_hbm.at[pl.ds(w * ECW + q * QC, QC), :], sv,
                         isem)
        pltpu.async_copy(dst_hbm.at[pl.ds(w * ECW + q * QC, QC), :], dv,
                         isem)

    def _refill_wait(p):
        sv, dv = idxsets[p]
        pltpu.make_async_copy(src_hbm.at[pl.ds(w * ECW, QC), :], sv,
                              isem).wait()
        pltpu.make_async_copy(dst_hbm.at[pl.ds(w * ECW, QC), :], dv,
                              isem).wait()

    _refill(0, 0)
    _refill_wait(0)
    _refill(1, 1)
    pltpu.async_copy(hp_hbm.at[src0_v.at[0]], ras, gsa)
    for q in range(NSEG):
        sv, dv = idxsets[q % 2]

        def body(i, carry, sv=sv, dv=dv, first=(q == 0)):
            def _wait_prev_b():
                pltpu.make_async_copy(rbs, acc.at[dv.at[0]], ssb).wait()

            if first:
                @pl.when(i > 0)
                def _():
                    _wait_prev_b()
            else:
                _wait_prev_b()
            pltpu.make_async_copy(hp_hbm.at[sv.at[2 * i]], ras, gsa).wait()
            pltpu.async_copy(ras, acc.at[dv.at[2 * i]], ssa, add=True)
            dgb = pltpu.async_copy(hp_hbm.at[sv.at[2 * i + 1]], rbs, gsb)
            dgb.wait()
            pltpu.make_async_copy(ras, acc.at[dv.at[2 * i]], ssa).wait()
            pltpu.async_copy(rbs, acc.at[dv.at[2 * i + 1]], ssb, add=True)

            @pl.when(i < QC // 2 - 1)
            def _():
                pltpu.async_copy(hp_hbm.at[sv.at[2 * i + 2]], ras, gsa)

            return carry

        lax.fori_loop(0, QC // 2, body, 0)
        if q + 1 < NSEG:
            nsv, _ndv = idxsets[(q + 1) % 2]
            _refill_wait((q + 1) % 2)
            pltpu.async_copy(hp_hbm.at[nsv.at[0]], ras, gsa)
            if q + 2 < NSEG:
                _refill(q + 2, q % 2)
        else:
            pltpu.make_async_copy(rbs, acc.at[dv.at[0]], ssb).wait()
    plsc.subcore_barrier()

    def _wb(out_hbm):
        def _ld2(j, buf, sem):
            return pltpu.async_copy(acc.at[pl.ds(s * RPT + j * CH, CH), :],
                                    buf, sem)

        dl2 = {0: _ld2(0, ra, gsa), 1: _ld2(1, rb_, gsb)}
        for j in range(RCH):
            buf, gsem, ssem = ((ra, gsa, ssa) if j % 2 == 0
                               else (rb_, gsb, ssb))
            dl2[j].wait()
            pltpu.async_copy(buf, out_hbm.at[pl.ds(s * RPT + j * CH, CH), :],
                             ssem).wait()
            if j + 2 < RCH:
                dl2[j + 2] = _ld2(j + 2, buf, gsem)

    @pl.when(c == 0)
    def _():
        _wb(out0_hbm)

    @pl.when(c == 1)
    def _():
        _wb(out1_hbm)


def _agg_call(hp, srcp, dstp):
    k = pl.kernel(
        _agg_body,
        out_type=(
            jax.ShapeDtypeStruct((NPAD, FD), jnp.float32),
            jax.ShapeDtypeStruct((NPAD, FD), jnp.float32),
        ),
        mesh=_mesh(),
        scratch_types=[
            pltpu.VMEM((QC, EC), jnp.int32),
            pltpu.VMEM((QC, EC), jnp.int32),
            pltpu.VMEM((QC, EC), jnp.int32),
            pltpu.VMEM((QC, EC), jnp.int32),
            pltpu.VMEM((CH, FD), jnp.float32),
            pltpu.VMEM((CH, FD), jnp.float32),
            pltpu.SemaphoreType.DMA,
            pltpu.SemaphoreType.DMA,
            pltpu.SemaphoreType.DMA,
            pltpu.SemaphoreType.DMA,
            pltpu.SemaphoreType.DMA,
            pltpu.VMEM_SHARED((NPAD, FD), jnp.float32),
        ],
    )
    return k(hp, srcp, dstp)


BR = 512
TGRID = NPAD // BR


def _dis(d0_ref, d1_ref):
    return lax.rsqrt(d0_ref[...] + d1_ref[...] + 1.0)


def _scale_body(x_ref, w_ref, d0_ref, d1_ref, hp_ref):
    h = jnp.dot(x_ref[...], w_ref[...], preferred_element_type=jnp.float32)
    hp_ref[...] = _dis(d0_ref, d1_ref) * h


def _scale_call(x, W1, deg0, deg1):
    blk = lambda i: (i, 0)
    return pl.pallas_call(
        _scale_body,
        grid=(TGRID,),
        in_specs=[
            pl.BlockSpec((BR, FD), blk),
            pl.BlockSpec((FD, FD), lambda i: (0, 0)),
            pl.BlockSpec((BR, 1), blk),
            pl.BlockSpec((BR, 1), blk),
        ],
        out_specs=pl.BlockSpec((BR, FD), blk),
        out_shape=jax.ShapeDtypeStruct((NPAD, FD), jnp.float32),
    )(x, W1, deg0, deg1)


def _ln_relu(z, g_ref, be_ref):
    ones_col = jnp.full((FD, 1), 1.0 / FD, jnp.float32)
    mu = jnp.dot(z, ones_col, preferred_element_type=jnp.float32)
    s2 = jnp.dot(z * z, ones_col, preferred_element_type=jnp.float32)
    rs = lax.rsqrt(s2 - mu * mu + EPS)
    g = g_ref[...]
    amat = jnp.dot(rs, g, preferred_element_type=jnp.float32)
    cmat = jnp.dot(mu * rs, g, preferred_element_type=jnp.float32) - be_ref[...]
    return jnp.maximum(z * amat - cmat, 0.0)


def _mid_body(a0_ref, a1_ref, hp_ref, d0_ref, d1_ref, b_ref, g_ref, be_ref,
              w2_ref, out_ref):
    dis = _dis(d0_ref, d1_ref)
    z = dis * (a0_ref[...] + a1_ref[...] - hp_ref[...]) + b_ref[...]
    r = _ln_relu(z, g_ref, be_ref)
    h2 = jnp.dot(r, w2_ref[...], preferred_element_type=jnp.float32)
    out_ref[...] = dis * h2


def _mid_call(a0, a1, hp, deg0, deg1, b1, g1, be1, W2):
    blk = lambda i: (i, 0)
    vec = lambda i: (0, 0)
    return pl.pallas_call(
        _mid_body,
        grid=(TGRID,),
        in_specs=[
            pl.BlockSpec((BR, FD), blk),
            pl.BlockSpec((BR, FD), blk),
            pl.BlockSpec((BR, FD), blk),
            pl.BlockSpec((BR, 1), blk),
            pl.BlockSpec((BR, 1), blk),
            pl.BlockSpec((1, FD), vec),
            pl.BlockSpec((1, FD), vec),
            pl.BlockSpec((1, FD), vec),
            pl.BlockSpec((FD, FD), vec),
        ],
        out_specs=pl.BlockSpec((BR, FD), blk),
        out_shape=jax.ShapeDtypeStruct((NPAD, FD), jnp.float32),
    )(a0, a1, hp, deg0, deg1, b1, g1, be1, W2)


def _final_body(a0_ref, a1_ref, hp_ref, d0_ref, d1_ref, b_ref, g_ref, be_ref,
                out_ref):
    i = pl.program_id(0)
    dis = _dis(d0_ref, d1_ref)
    z = dis * (a0_ref[...] + a1_ref[...] - hp_ref[...]) + b_ref[...]
    r = _ln_relu(z, g_ref, be_ref)
    rowid = lax.broadcasted_iota(jnp.int32, (BR, FD), 0) + i * BR
    r = jnp.where(rowid < NN, r, 0.0)
    part = jnp.dot(jnp.ones((1, BR), jnp.float32), r,
                   preferred_element_type=jnp.float32)

    @pl.when(i == 0)
    def _():
        out_ref[...] = jnp.zeros((1, FD), jnp.float32)

    out_ref[...] += part

    @pl.when(i == TGRID - 1)
    def _():
        out_ref[...] = out_ref[...] * (1.0 / NN)


def _final_call(a0, a1, hp, deg0, deg1, b2, g2, be2):
    blk = lambda i: (i, 0)
    vec = lambda i: (0, 0)
    return pl.pallas_call(
        _final_body,
        grid=(TGRID,),
        in_specs=[
            pl.BlockSpec((BR, FD), blk),
            pl.BlockSpec((BR, FD), blk),
            pl.BlockSpec((BR, FD), blk),
            pl.BlockSpec((BR, 1), blk),
            pl.BlockSpec((BR, 1), blk),
            pl.BlockSpec((1, FD), vec),
            pl.BlockSpec((1, FD), vec),
            pl.BlockSpec((1, FD), vec),
        ],
        out_specs=pl.BlockSpec((1, FD), vec),
        out_shape=jax.ShapeDtypeStruct((1, FD), jnp.float32),
    )(a0, a1, hp, deg0, deg1, b2, g2, be2)


def kernel(x, edge_index, W1, b1, g1, be1, W2, b2, g2, be2):
    srcp = edge_index[0].astype(jnp.int32).reshape(NW * ECW, EC)
    dstp = edge_index[1].astype(jnp.int32).reshape(NW * ECW, EC)

    deg0, deg1 = _deg_call(dstp)
    deg0 = deg0.reshape(NPAD, 1)
    deg1 = deg1.reshape(NPAD, 1)
    hp1 = _scale_call(x, W1, deg0, deg1)
    a10, a11 = _agg_call(hp1, srcp, dstp)
    hp2 = _mid_call(a10, a11, hp1, deg0, deg1,
                    b1.reshape(1, FD), g1.reshape(1, FD), be1.reshape(1, FD),
                    W2)
    a20, a21 = _agg_call(hp2, srcp, dstp)
    return _final_call(a20, a21, hp2, deg0, deg1,
                       b2.reshape(1, FD), g2.reshape(1, FD),
                       be2.reshape(1, FD))


# speedup vs baseline: 1.0063x; 1.0063x over previous
import functools

import jax
import jax.numpy as jnp
from jax import lax
from jax.experimental import pallas as pl
from jax.experimental.pallas import tpu as pltpu
from jax.experimental.pallas import tpu_sc as plsc

NN = 10000
FD = 128
NE = 320000
NC = 2
NS = 16
NW = NC * NS
NPAD = 10240
RPT = NPAD // NS
CH = 128
RCH = RPT // CH
EC = 125
ECW = NE // NW // EC
QC = 16
NSEG = ECW // QC
EPS = 1e-5


def _mesh():
    return plsc.VectorSubcoreMesh(core_axis_name="c", subcore_axis_name="s")


def _deg_body(dst_hbm, out0_hbm, out1_hbm, dsts_v, ones_v, stg1,
              sem0, sem1, sem2, sem3, acc):
    c = lax.axis_index("c")
    s = lax.axis_index("s")
    w = c * NS + s
    slab = pl.ds(s * RPT, RPT)
    for t in range(128 // 16):
        ones_v[pl.ds(16 * t, 16)] = jnp.ones((16,), jnp.float32)
    for t in range(RPT // 16):
        stg1[pl.ds(16 * t, 16)] = jnp.zeros((16,), jnp.float32)
    pltpu.sync_copy(stg1, acc.at[slab])
    pltpu.sync_copy(dst_hbm.at[pl.ds(w * ECW, ECW), :], dsts_v)
    plsc.subcore_barrier()
    sems = [sem0, sem1, sem2, sem3]

    def body(i, carry):
        descs = []
        for b in range(4):
            descs.append(pltpu.async_copy(
                ones_v.at[pl.ds(0, EC)], acc.at[dsts_v.at[i * 4 + b]],
                sems[b], add=True))
        for d in descs:
            d.wait()
        return carry

    lax.fori_loop(0, ECW // 4, body, 0)
    plsc.subcore_barrier()
    pltpu.sync_copy(acc.at[slab], stg1)

    @pl.when(c == 0)
    def _():
        pltpu.sync_copy(stg1, out0_hbm.at[slab])

    @pl.when(c == 1)
    def _():
        pltpu.sync_copy(stg1, out1_hbm.at[slab])


def _deg_call(dstp):
    k = pl.kernel(
        _deg_body,
        out_type=(
            jax.ShapeDtypeStruct((NPAD,), jnp.float32),
            jax.ShapeDtypeStruct((NPAD,), jnp.float32),
        ),
        mesh=_mesh(),
        scratch_types=[
            pltpu.VMEM((ECW, EC), jnp.int32),
            pltpu.VMEM((128,), jnp.float32),
            pltpu.VMEM((RPT,), jnp.float32),
            pltpu.SemaphoreType.DMA,
            pltpu.SemaphoreType.DMA,
            pltpu.SemaphoreType.DMA,
            pltpu.SemaphoreType.DMA,
            pltpu.VMEM_SHARED((NPAD,), jnp.float32),
        ],
    )
    return k(dstp)


def _agg_body(hp_hbm, src_hbm, dst_hbm, out0_hbm, out1_hbm,
              src0_v, dst0_v, src1_v, dst1_v,
              ra, rb_, gsa, gsb, ssa, ssb, isem, acc):
    c = lax.axis_index("c")
    s = lax.axis_index("s")
    w = c * NS + s

    def _ld(j, buf, sem):
        return pltpu.async_copy(hp_hbm.at[pl.ds(s * RPT + j * CH, CH), :],
                                buf, sem)

    dl = {0: _ld(0, ra, gsa), 1: _ld(1, rb_, gsb)}
    for j in range(RCH):
        buf, gsem, ssem = (ra, gsa, ssa) if j % 2 == 0 else (rb_, gsb, ssb)
        dl[j].wait()
        pltpu.async_copy(buf, acc.at[pl.ds(s * RPT + j * CH, CH), :],
                         ssem).wait()
        if j + 2 < RCH:
            dl[j + 2] = _ld(j + 2, buf, gsem)
    plsc.subcore_barrier()

    ras = ra.at[pl.ds(0, EC), :]
    rbs = rb_.at[pl.ds(0, EC), :]
    idxsets = [(src0_v, dst0_v), (src1_v, dst1_v)]

    def _refill(q, p):
        sv, dv = idxsets[p]
        pltpu.async_copy(src_hbm.at[pl.ds(w * ECW + q * QC, QC), :], sv,
                         isem)
        pltpu.async_copy(dst_hbm.at[pl.ds(w * ECW + q * QC, QC), :], dv,
                         isem)

    def _refill_wait(p):
        sv, dv = idxsets[p]
        pltpu.make_async_copy(src_hbm.at[pl.ds(w * ECW, QC), :], sv,
                              isem).wait()
        pltpu.make_async_copy(dst_hbm.at[pl.ds(w * ECW, QC), :], dv,
                              isem).wait()

    _refill(0, 0)
    _refill_wait(0)
    _refill(1, 1)
    pltpu.async_copy(hp_hbm.at[src0_v.at[0]], ras, gsa)
    for q in range(NSEG):
        sv, dv = idxsets[q % 2]

        def body(i, carry, sv=sv, dv=dv, first=(q == 0)):
            def _wait_prev_b():
                pltpu.make_async_copy(rbs, acc.at[dv.at[0]], ssb).wait()

            if first:
                @pl.when(i > 0)
                def _():
                    _wait_prev_b()
            else:
                _wait_prev_b()
            pltpu.make_async_copy(hp_hbm.at[sv.at[2 * i]], ras, gsa).wait()
            pltpu.async_copy(ras, acc.at[dv.at[2 * i]], ssa, add=True)
            dgb = pltpu.async_copy(hp_hbm.at[sv.at[2 * i + 1]], rbs, gsb)
            dgb.wait()
            pltpu.async_copy(rbs, acc.at[dv.at[2 * i + 1]], ssb, add=True)
            pltpu.make_async_copy(ras, acc.at[dv.at[2 * i]], ssa).wait()

            @pl.when(i < QC // 2 - 1)
            def _():
                pltpu.async_copy(hp_hbm.at[sv.at[2 * i + 2]], ras, gsa)

            return carry

        lax.fori_loop(0, QC // 2, body, 0)
        if q + 1 < NSEG:
            nsv, _ndv = idxsets[(q + 1) % 2]
            _refill_wait((q + 1) % 2)
            pltpu.async_copy(hp_hbm.at[nsv.at[0]], ras, gsa)
            if q + 2 < NSEG:
                _refill(q + 2, q % 2)
        else:
            pltpu.make_async_copy(rbs, acc.at[dv.at[0]], ssb).wait()
    plsc.subcore_barrier()

    def _wb(out_hbm):
        def _ld2(j, buf, sem):
            return pltpu.async_copy(acc.at[pl.ds(s * RPT + j * CH, CH), :],
                                    buf, sem)

        dl2 = {0: _ld2(0, ra, gsa), 1: _ld2(1, rb_, gsb)}
        for j in range(RCH):
            buf, gsem, ssem = ((ra, gsa, ssa) if j % 2 == 0
                               else (rb_, gsb, ssb))
            dl2[j].wait()
            pltpu.async_copy(buf, out_hbm.at[pl.ds(s * RPT + j * CH, CH), :],
                             ssem).wait()
            if j + 2 < RCH:
                dl2[j + 2] = _ld2(j + 2, buf, gsem)

    @pl.when(c == 0)
    def _():
        _wb(out0_hbm)

    @pl.when(c == 1)
    def _():
        _wb(out1_hbm)


def _agg_call(hp, srcp, dstp):
    k = pl.kernel(
        _agg_body,
        out_type=(
            jax.ShapeDtypeStruct((NPAD, FD), jnp.float32),
            jax.ShapeDtypeStruct((NPAD, FD), jnp.float32),
        ),
        mesh=_mesh(),
        scratch_types=[
            pltpu.VMEM((QC, EC), jnp.int32),
            pltpu.VMEM((QC, EC), jnp.int32),
            pltpu.VMEM((QC, EC), jnp.int32),
            pltpu.VMEM((QC, EC), jnp.int32),
            pltpu.VMEM((CH, FD), jnp.float32),
            pltpu.VMEM((CH, FD), jnp.float32),
            pltpu.SemaphoreType.DMA,
            pltpu.SemaphoreType.DMA,
            pltpu.SemaphoreType.DMA,
            pltpu.SemaphoreType.DMA,
            pltpu.SemaphoreType.DMA,
            pltpu.VMEM_SHARED((NPAD, FD), jnp.float32),
        ],
    )
    return k(hp, srcp, dstp)


BR = 512
TGRID = NPAD // BR


def _dis(d0_ref, d1_ref):
    return lax.rsqrt(d0_ref[...] + d1_ref[...] + 1.0)


def _scale_body(x_ref, w_ref, d0_ref, d1_ref, hp_ref):
    h = jnp.dot(x_ref[...], w_ref[...], preferred_element_type=jnp.float32)
    hp_ref[...] = _dis(d0_ref, d1_ref) * h


def _scale_call(x, W1, deg0, deg1):
    blk = lambda i: (i, 0)
    return pl.pallas_call(
        _scale_body,
        grid=(TGRID,),
        in_specs=[
            pl.BlockSpec((BR, FD), blk),
            pl.BlockSpec((FD, FD), lambda i: (0, 0)),
            pl.BlockSpec((BR, 1), blk),
            pl.BlockSpec((BR, 1), blk),
        ],
        out_specs=pl.BlockSpec((BR, FD), blk),
        out_shape=jax.ShapeDtypeStruct((NPAD, FD), jnp.float32),
    )(x, W1, deg0, deg1)


def _ln_relu(z, g_ref, be_ref):
    ones_col = jnp.full((FD, 1), 1.0 / FD, jnp.float32)
    mu = jnp.dot(z, ones_col, preferred_element_type=jnp.float32)
    s2 = jnp.dot(z * z, ones_col, preferred_element_type=jnp.float32)
    rs = lax.rsqrt(s2 - mu * mu + EPS)
    g = g_ref[...]
    amat = jnp.dot(rs, g, preferred_element_type=jnp.float32)
    cmat = jnp.dot(mu * rs, g, preferred_element_type=jnp.float32) - be_ref[...]
    return jnp.maximum(z * amat - cmat, 0.0)


def _mid_body(a0_ref, a1_ref, hp_ref, d0_ref, d1_ref, b_ref, g_ref, be_ref,
              w2_ref, out_ref):
    dis = _dis(d0_ref, d1_ref)
    z = dis * (a0_ref[...] + a1_ref[...] - hp_ref[...]) + b_ref[...]
    r = _ln_relu(z, g_ref, be_ref)
    h2 = jnp.dot(r, w2_ref[...], preferred_element_type=jnp.float32)
    out_ref[...] = dis * h2


def _mid_call(a0, a1, hp, deg0, deg1, b1, g1, be1, W2):
    blk = lambda i: (i, 0)
    vec = lambda i: (0, 0)
    return pl.pallas_call(
        _mid_body,
        grid=(TGRID,),
        in_specs=[
            pl.BlockSpec((BR, FD), blk),
            pl.BlockSpec((BR, FD), blk),
            pl.BlockSpec((BR, FD), blk),
            pl.BlockSpec((BR, 1), blk),
            pl.BlockSpec((BR, 1), blk),
            pl.BlockSpec((1, FD), vec),
            pl.BlockSpec((1, FD), vec),
            pl.BlockSpec((1, FD), vec),
            pl.BlockSpec((FD, FD), vec),
        ],
        out_specs=pl.BlockSpec((BR, FD), blk),
        out_shape=jax.ShapeDtypeStruct((NPAD, FD), jnp.float32),
    )(a0, a1, hp, deg0, deg1, b1, g1, be1, W2)


def _final_body(a0_ref, a1_ref, hp_ref, d0_ref, d1_ref, b_ref, g_ref, be_ref,
                out_ref):
    i = pl.program_id(0)
    dis = _dis(d0_ref, d1_ref)
    z = dis * (a0_ref[...] + a1_ref[...] - hp_ref[...]) + b_ref[...]
    r = _ln_relu(z, g_ref, be_ref)
    rowid = lax.broadcasted_iota(jnp.int32, (BR, FD), 0) + i * BR
    r = jnp.where(rowid < NN, r, 0.0)
    part = jnp.dot(jnp.ones((1, BR), jnp.float32), r,
                   preferred_element_type=jnp.float32)

    @pl.when(i == 0)
    def _():
        out_ref[...] = jnp.zeros((1, FD), jnp.float32)

    out_ref[...] += part

    @pl.when(i == TGRID - 1)
    def _():
        out_ref[...] = out_ref[...] * (1.0 / NN)


def _final_call(a0, a1, hp, deg0, deg1, b2, g2, be2):
    blk = lambda i: (i, 0)
    vec = lambda i: (0, 0)
    return pl.pallas_call(
        _final_body,
        grid=(TGRID,),
        in_specs=[
            pl.BlockSpec((BR, FD), blk),
            pl.BlockSpec((BR, FD), blk),
            pl.BlockSpec((BR, FD), blk),
            pl.BlockSpec((BR, 1), blk),
            pl.BlockSpec((BR, 1), blk),
            pl.BlockSpec((1, FD), vec),
            pl.BlockSpec((1, FD), vec),
            pl.BlockSpec((1, FD), vec),
        ],
        out_specs=pl.BlockSpec((1, FD), vec),
        out_shape=jax.ShapeDtypeStruct((1, FD), jnp.float32),
    )(a0, a1, hp, deg0, deg1, b2, g2, be2)


def kernel(x, edge_index, W1, b1, g1, be1, W2, b2, g2, be2):
    srcp = edge_index[0].astype(jnp.int32).reshape(NW * ECW, EC)
    dstp = edge_index[1].astype(jnp.int32).reshape(NW * ECW, EC)

    deg0, deg1 = _deg_call(dstp)
    deg0 = deg0.reshape(NPAD, 1)
    deg1 = deg1.reshape(NPAD, 1)
    hp1 = _scale_call(x, W1, deg0, deg1)
    a10, a11 = _agg_call(hp1, srcp, dstp)
    hp2 = _mid_call(a10, a11, hp1, deg0, deg1,
                    b1.reshape(1, FD), g1.reshape(1, FD), be1.reshape(1, FD),
                    W2)
    a20, a21 = _agg_call(hp2, srcp, dstp)
    return _final_call(a20, a21, hp2, deg0, deg1,
                       b2.reshape(1, FD), g2.reshape(1, FD),
                       be2.reshape(1, FD))


# speedup vs baseline: 1.0239x; 1.0175x over previous
import functools

import jax
import jax.numpy as jnp
from jax import lax
from jax.experimental import pallas as pl
from jax.experimental.pallas import tpu as pltpu
from jax.experimental.pallas import tpu_sc as plsc

NN = 10000
FD = 128
NE = 320000
NC = 2
NS = 16
NW = NC * NS
NPAD = 10240
RPT = NPAD // NS
CH = 128
RCH = RPT // CH
EC = 125
ECW = NE // NW // EC
QC = 16
NSEG = ECW // QC
EPS = 1e-5


def _mesh():
    return plsc.VectorSubcoreMesh(core_axis_name="c", subcore_axis_name="s")


def _deg_body(dst_hbm, out0_hbm, out1_hbm, dsts_v, ones_v, stg1,
              sem0, sem1, sem2, sem3, acc):
    c = lax.axis_index("c")
    s = lax.axis_index("s")
    w = c * NS + s
    slab = pl.ds(s * RPT, RPT)
    for t in range(128 // 16):
        ones_v[pl.ds(16 * t, 16)] = jnp.ones((16,), jnp.float32)
    for t in range(RPT // 16):
        stg1[pl.ds(16 * t, 16)] = jnp.zeros((16,), jnp.float32)
    pltpu.sync_copy(stg1, acc.at[slab])
    pltpu.sync_copy(dst_hbm.at[pl.ds(w * ECW, ECW), :], dsts_v)
    plsc.subcore_barrier()
    sems = [sem0, sem1, sem2, sem3]

    def body(i, carry):
        descs = []
        for b in range(4):
            descs.append(pltpu.async_copy(
                ones_v.at[pl.ds(0, EC)], acc.at[dsts_v.at[i * 4 + b]],
                sems[b], add=True))
        for d in descs:
            d.wait()
        return carry

    lax.fori_loop(0, ECW // 4, body, 0)
    plsc.subcore_barrier()
    pltpu.sync_copy(acc.at[slab], stg1)

    @pl.when(c == 0)
    def _():
        pltpu.sync_copy(stg1, out0_hbm.at[slab])

    @pl.when(c == 1)
    def _():
        pltpu.sync_copy(stg1, out1_hbm.at[slab])


def _deg_call(dstp):
    k = pl.kernel(
        _deg_body,
        out_type=(
            jax.ShapeDtypeStruct((NPAD,), jnp.float32),
            jax.ShapeDtypeStruct((NPAD,), jnp.float32),
        ),
        mesh=_mesh(),
        scratch_types=[
            pltpu.VMEM((ECW, EC), jnp.int32),
            pltpu.VMEM((128,), jnp.float32),
            pltpu.VMEM((RPT,), jnp.float32),
            pltpu.SemaphoreType.DMA,
            pltpu.SemaphoreType.DMA,
            pltpu.SemaphoreType.DMA,
            pltpu.SemaphoreType.DMA,
            pltpu.VMEM_SHARED((NPAD,), jnp.float32),
        ],
    )
    return k(dstp)


def _agg_body(hp_hbm, src_hbm, dst_hbm, out0_hbm, out1_hbm,
              src0_v, dst0_v, src1_v, dst1_v,
              ra, rb_, gsa, gsb, ssa, ssb, isem, acc):
    c = lax.axis_index("c")
    s = lax.axis_index("s")
    w = c * NS + s

    def _ld(j, buf, sem):
        return pltpu.async_copy(hp_hbm.at[pl.ds(s * RPT + j * CH, CH), :],
                                buf, sem)

    dl = {0: _ld(0, ra, gsa), 1: _ld(1, rb_, gsb)}
    for j in range(RCH):
        buf, gsem, ssem = (ra, gsa, ssa) if j % 2 == 0 else (rb_, gsb, ssb)
        dl[j].wait()
        pltpu.async_copy(buf, acc.at[pl.ds(s * RPT + j * CH, CH), :],
                         ssem).wait()
        if j + 2 < RCH:
            dl[j + 2] = _ld(j + 2, buf, gsem)
    plsc.subcore_barrier()

    ras = ra.at[pl.ds(0, EC), :]
    rbs = rb_.at[pl.ds(0, EC), :]
    idxsets = [(src0_v, dst0_v), (src1_v, dst1_v)]

    def _refill(q, p):
        sv, dv = idxsets[p]
        pltpu.async_copy(src_hbm.at[pl.ds(w * ECW + q * QC, QC), :], sv,
                         isem)
        pltpu.async_copy(dst_hbm.at[pl.ds(w * ECW + q * QC, QC), :], dv,
                         isem)

    def _refill_wait(p):
        sv, dv = idxsets[p]
        pltpu.make_async_copy(src_hbm.at[pl.ds(w * ECW, QC), :], sv,
                              isem).wait()
        pltpu.make_async_copy(dst_hbm.at[pl.ds(w * ECW, QC), :], dv,
                              isem).wait()

    _refill(0, 0)
    _refill_wait(0)
    _refill(1, 1)
    pltpu.async_copy(hp_hbm.at[src0_v.at[0]], ras, gsa)
    for q in range(NSEG):
        sv, dv = idxsets[q % 2]

        def body(i, carry, sv=sv, dv=dv, first=(q == 0)):
            def _wait_prev_b():
                pltpu.make_async_copy(rbs, acc.at[dv.at[0]], ssb).wait()

            if first:
                @pl.when(i > 0)
                def _():
                    _wait_prev_b()
            else:
                _wait_prev_b()
            pltpu.make_async_copy(hp_hbm.at[sv.at[2 * i]], ras, gsa).wait()
            pltpu.async_copy(ras, acc.at[dv.at[2 * i]], ssa, add=True)
            dgb = pltpu.async_copy(hp_hbm.at[sv.at[2 * i + 1]], rbs, gsb)
            dgb.wait()
            pltpu.async_copy(rbs, acc.at[dv.at[2 * i + 1]], ssb, add=True)
            pltpu.make_async_copy(ras, acc.at[dv.at[2 * i]], ssa).wait()

            @pl.when(i < QC // 2 - 1)
            def _():
                pltpu.async_copy(hp_hbm.at[sv.at[2 * i + 2]], ras, gsa)

            return carry

        lax.fori_loop(0, QC // 2, body, 0)
        if q + 1 < NSEG:
            nsv, _ndv = idxsets[(q + 1) % 2]
            _refill_wait((q + 1) % 2)
            pltpu.async_copy(hp_hbm.at[nsv.at[0]], ras, gsa)
            if q + 2 < NSEG:
                _refill(q + 2, q % 2)
        else:
            pltpu.make_async_copy(rbs, acc.at[dv.at[0]], ssb).wait()
    plsc.subcore_barrier()

    def _wb(out_hbm):
        def _ld2(j, buf, sem):
            return pltpu.async_copy(acc.at[pl.ds(s * RPT + j * CH, CH), :],
                                    buf, sem)

        dl2 = {0: _ld2(0, ra, gsa), 1: _ld2(1, rb_, gsb)}
        for j in range(RCH):
            buf, gsem, ssem = ((ra, gsa, ssa) if j % 2 == 0
                               else (rb_, gsb, ssb))
            dl2[j].wait()
            pltpu.async_copy(buf, out_hbm.at[pl.ds(s * RPT + j * CH, CH), :],
                             ssem).wait()
            if j + 2 < RCH:
                dl2[j + 2] = _ld2(j + 2, buf, gsem)

    @pl.when(c == 0)
    def _():
        _wb(out0_hbm)

    @pl.when(c == 1)
    def _():
        _wb(out1_hbm)


def _agg_call(hp, srcp, dstp):
    k = pl.kernel(
        _agg_body,
        out_type=(
            jax.ShapeDtypeStruct((NPAD, FD), jnp.float32),
            jax.ShapeDtypeStruct((NPAD, FD), jnp.float32),
        ),
        mesh=_mesh(),
        scratch_types=[
            pltpu.VMEM((QC, EC), jnp.int32),
            pltpu.VMEM((QC, EC), jnp.int32),
            pltpu.VMEM((QC, EC), jnp.int32),
            pltpu.VMEM((QC, EC), jnp.int32),
            pltpu.VMEM((CH, FD), jnp.float32),
            pltpu.VMEM((CH, FD), jnp.float32),
            pltpu.SemaphoreType.DMA,
            pltpu.SemaphoreType.DMA,
            pltpu.SemaphoreType.DMA,
            pltpu.SemaphoreType.DMA,
            pltpu.SemaphoreType.DMA,
            pltpu.VMEM_SHARED((NPAD, FD), jnp.float32),
        ],
    )
    return k(hp, srcp, dstp)


BR = 512
TGRID = NPAD // BR


def _dis(d_ref):
    return lax.rsqrt(d_ref[...] + 1.0)


def _scale_body(x_ref, w_ref, d_ref, hp_ref):
    h = jnp.dot(x_ref[...], w_ref[...], preferred_element_type=jnp.float32)
    hp_ref[...] = _dis(d_ref) * h


def _scale_call(x, W1, degs):
    blk = lambda i: (i, 0)
    return pl.pallas_call(
        _scale_body,
        grid=(TGRID,),
        in_specs=[
            pl.BlockSpec((BR, FD), blk),
            pl.BlockSpec((FD, FD), lambda i: (0, 0)),
            pl.BlockSpec((BR, 1), blk),
        ],
        out_specs=pl.BlockSpec((BR, FD), blk),
        out_shape=jax.ShapeDtypeStruct((NPAD, FD), jnp.float32),
    )(x, W1, degs)


def _ln_relu(z, g_ref, be_ref):
    ones_col = jnp.full((FD, 1), 1.0 / FD, jnp.float32)
    mu = jnp.dot(z, ones_col, preferred_element_type=jnp.float32)
    s2 = jnp.dot(z * z, ones_col, preferred_element_type=jnp.float32)
    rs = lax.rsqrt(s2 - mu * mu + EPS)
    g = g_ref[...]
    amat = jnp.dot(rs, g, preferred_element_type=jnp.float32)
    cmat = jnp.dot(mu * rs, g, preferred_element_type=jnp.float32) - be_ref[...]
    return jnp.maximum(z * amat - cmat, 0.0)


def _mid_body(a0_ref, a1_ref, hp_ref, d_ref, b_ref, g_ref, be_ref,
              w2_ref, out_ref):
    dis = _dis(d_ref)
    z = dis * (a0_ref[...] + a1_ref[...] - hp_ref[...]) + b_ref[...]
    r = _ln_relu(z, g_ref, be_ref)
    h2 = jnp.dot(r, w2_ref[...], preferred_element_type=jnp.float32)
    out_ref[...] = dis * h2


def _mid_call(a0, a1, hp, degs, b1, g1, be1, W2):
    blk = lambda i: (i, 0)
    vec = lambda i: (0, 0)
    return pl.pallas_call(
        _mid_body,
        grid=(TGRID,),
        in_specs=[
            pl.BlockSpec((BR, FD), blk),
            pl.BlockSpec((BR, FD), blk),
            pl.BlockSpec((BR, FD), blk),
            pl.BlockSpec((BR, 1), blk),
            pl.BlockSpec((1, FD), vec),
            pl.BlockSpec((1, FD), vec),
            pl.BlockSpec((1, FD), vec),
            pl.BlockSpec((FD, FD), vec),
        ],
        out_specs=pl.BlockSpec((BR, FD), blk),
        out_shape=jax.ShapeDtypeStruct((NPAD, FD), jnp.float32),
    )(a0, a1, hp, degs, b1, g1, be1, W2)


def _final_body(a0_ref, a1_ref, hp_ref, d_ref, b_ref, g_ref, be_ref,
                out_ref):
    i = pl.program_id(0)
    dis = _dis(d_ref)
    z = dis * (a0_ref[...] + a1_ref[...] - hp_ref[...]) + b_ref[...]
    r = _ln_relu(z, g_ref, be_ref)
    rowid = lax.broadcasted_iota(jnp.int32, (BR, FD), 0) + i * BR
    r = jnp.where(rowid < NN, r, 0.0)
    part = jnp.dot(jnp.ones((1, BR), jnp.float32), r,
                   preferred_element_type=jnp.float32)

    @pl.when(i == 0)
    def _():
        out_ref[...] = jnp.zeros((1, FD), jnp.float32)

    out_ref[...] += part

    @pl.when(i == TGRID - 1)
    def _():
        out_ref[...] = out_ref[...] * (1.0 / NN)


def _final_call(a0, a1, hp, degs, b2, g2, be2):
    blk = lambda i: (i, 0)
    vec = lambda i: (0, 0)
    return pl.pallas_call(
        _final_body,
        grid=(TGRID,),
        in_specs=[
            pl.BlockSpec((BR, FD), blk),
            pl.BlockSpec((BR, FD), blk),
            pl.BlockSpec((BR, FD), blk),
            pl.BlockSpec((BR, 1), blk),
            pl.BlockSpec((1, FD), vec),
            pl.BlockSpec((1, FD), vec),
            pl.BlockSpec((1, FD), vec),
        ],
        out_specs=pl.BlockSpec((1, FD), vec),
        out_shape=jax.ShapeDtypeStruct((1, FD), jnp.float32),
    )(a0, a1, hp, degs, b2, g2, be2)


def kernel(x, edge_index, W1, b1, g1, be1, W2, b2, g2, be2):
    srcp = edge_index[0].astype(jnp.int32).reshape(NW * ECW, EC)
    dstp = edge_index[1].astype(jnp.int32).reshape(NW * ECW, EC)

    deg0, deg1 = _deg_call(dstp)
    degs = (deg0 + deg1).reshape(NPAD, 1)
    hp1 = _scale_call(x, W1, degs)
    a10, a11 = _agg_call(hp1, srcp, dstp)
    hp2 = _mid_call(a10, a11, hp1, degs,
                    b1.reshape(1, FD), g1.reshape(1, FD), be1.reshape(1, FD),
                    W2)
    a20, a21 = _agg_call(hp2, srcp, dstp)
    return _final_call(a20, a21, hp2, degs,
                       b2.reshape(1, FD), g2.reshape(1, FD),
                       be2.reshape(1, FD))


# speedup vs baseline: 1.0255x; 1.0015x over previous
import functools

import jax
import jax.numpy as jnp
from jax import lax
from jax.experimental import pallas as pl
from jax.experimental.pallas import tpu as pltpu
from jax.experimental.pallas import tpu_sc as plsc

NN = 10000
FD = 128
NE = 320000
NC = 2
NS = 16
NW = NC * NS
NPAD = 10240
RPT = NPAD // NS
CH = 128
RCH = RPT // CH
EC = 125
ECW = NE // NW // EC
QC = 16
NSEG = ECW // QC
EPS = 1e-5


def _mesh():
    return plsc.VectorSubcoreMesh(core_axis_name="c", subcore_axis_name="s")


def _deg_body(dst_hbm, out0_hbm, out1_hbm, dsts_v, ones_v, stg1,
              sem0, sem1, sem2, sem3, acc):
    c = lax.axis_index("c")
    s = lax.axis_index("s")
    w = c * NS + s
    slab = pl.ds(s * RPT, RPT)
    for t in range(128 // 16):
        ones_v[pl.ds(16 * t, 16)] = jnp.ones((16,), jnp.float32)
    for t in range(RPT // 16):
        stg1[pl.ds(16 * t, 16)] = jnp.zeros((16,), jnp.float32)
    pltpu.sync_copy(stg1, acc.at[slab])
    pltpu.sync_copy(dst_hbm.at[pl.ds(w * ECW, ECW), :], dsts_v)
    plsc.subcore_barrier()
    sems = [sem0, sem1, sem2, sem3]

    def body(i, carry):
        descs = []
        for b in range(4):
            descs.append(pltpu.async_copy(
                ones_v.at[pl.ds(0, EC)], acc.at[dsts_v.at[i * 4 + b]],
                sems[b], add=True))
        for d in descs:
            d.wait()
        return carry

    lax.fori_loop(0, ECW // 4, body, 0)
    plsc.subcore_barrier()
    pltpu.sync_copy(acc.at[slab], stg1)

    @pl.when(c == 0)
    def _():
        pltpu.sync_copy(stg1, out0_hbm.at[slab])

    @pl.when(c == 1)
    def _():
        pltpu.sync_copy(stg1, out1_hbm.at[slab])


def _deg_call(dstp):
    k = pl.kernel(
        _deg_body,
        out_type=(
            jax.ShapeDtypeStruct((NPAD,), jnp.float32),
            jax.ShapeDtypeStruct((NPAD,), jnp.float32),
        ),
        mesh=_mesh(),
        scratch_types=[
            pltpu.VMEM((ECW, EC), jnp.int32),
            pltpu.VMEM((128,), jnp.float32),
            pltpu.VMEM((RPT,), jnp.float32),
            pltpu.SemaphoreType.DMA,
            pltpu.SemaphoreType.DMA,
            pltpu.SemaphoreType.DMA,
            pltpu.SemaphoreType.DMA,
            pltpu.VMEM_SHARED((NPAD,), jnp.float32),
        ],
    )
    return k(dstp)


def _agg_body(hp_hbm, src_hbm, dst_hbm, out0_hbm, out1_hbm,
              src0_v, dst0_v, src1_v, dst1_v,
              ra, rb_, gsa, gsb, ssa, ssb, isem0, isem1, acc):
    c = lax.axis_index("c")
    s = lax.axis_index("s")
    w = c * NS + s

    idxsets0 = [(src0_v, dst0_v), (src1_v, dst1_v)]
    isems0 = [isem0, isem1]

    def _refill0(q, p):
        sv, dv = idxsets0[p]
        pltpu.async_copy(src_hbm.at[pl.ds(w * ECW + q * QC, QC), :], sv,
                         isems0[p])
        pltpu.async_copy(dst_hbm.at[pl.ds(w * ECW + q * QC, QC), :], dv,
                         isems0[p])

    _refill0(0, 0)
    _refill0(1, 1)

    def _ld(j, buf, sem):
        return pltpu.async_copy(hp_hbm.at[pl.ds(s * RPT + j * CH, CH), :],
                                buf, sem)

    dl = {0: _ld(0, ra, gsa), 1: _ld(1, rb_, gsb)}
    for j in range(RCH):
        buf, gsem, ssem = (ra, gsa, ssa) if j % 2 == 0 else (rb_, gsb, ssb)
        dl[j].wait()
        pltpu.async_copy(buf, acc.at[pl.ds(s * RPT + j * CH, CH), :],
                         ssem).wait()
        if j + 2 < RCH:
            dl[j + 2] = _ld(j + 2, buf, gsem)
    plsc.subcore_barrier()

    ras = ra.at[pl.ds(0, EC), :]
    rbs = rb_.at[pl.ds(0, EC), :]
    idxsets = [(src0_v, dst0_v), (src1_v, dst1_v)]

    isems = [isem0, isem1]

    def _refill(q, p):
        sv, dv = idxsets[p]
        pltpu.async_copy(src_hbm.at[pl.ds(w * ECW + q * QC, QC), :], sv,
                         isems[p])
        pltpu.async_copy(dst_hbm.at[pl.ds(w * ECW + q * QC, QC), :], dv,
                         isems[p])

    def _refill_wait(p):
        sv, dv = idxsets[p]
        pltpu.make_async_copy(src_hbm.at[pl.ds(w * ECW, QC), :], sv,
                              isems[p]).wait()
        pltpu.make_async_copy(dst_hbm.at[pl.ds(w * ECW, QC), :], dv,
                              isems[p]).wait()

    _refill_wait(0)
    pltpu.async_copy(hp_hbm.at[src0_v.at[0]], ras, gsa)
    for q in range(NSEG):
        sv, dv = idxsets[q % 2]

        def body(i, carry, sv=sv, dv=dv, first=(q == 0)):
            def _wait_prev_b():
                pltpu.make_async_copy(rbs, acc.at[dv.at[0]], ssb).wait()

            if first:
                @pl.when(i > 0)
                def _():
                    _wait_prev_b()
            else:
                _wait_prev_b()
            pltpu.make_async_copy(hp_hbm.at[sv.at[2 * i]], ras, gsa).wait()
            pltpu.async_copy(ras, acc.at[dv.at[2 * i]], ssa, add=True)
            dgb = pltpu.async_copy(hp_hbm.at[sv.at[2 * i + 1]], rbs, gsb)
            dgb.wait()
            pltpu.async_copy(rbs, acc.at[dv.at[2 * i + 1]], ssb, add=True)
            pltpu.make_async_copy(ras, acc.at[dv.at[2 * i]], ssa).wait()

            @pl.when(i < QC // 2 - 1)
            def _():
                pltpu.async_copy(hp_hbm.at[sv.at[2 * i + 2]], ras, gsa)

            return carry

        lax.fori_loop(0, QC // 2, body, 0)
        if q + 1 < NSEG:
            nsv, _ndv = idxsets[(q + 1) % 2]
            _refill_wait((q + 1) % 2)
            pltpu.async_copy(hp_hbm.at[nsv.at[0]], ras, gsa)
            if q + 2 < NSEG:
                _refill(q + 2, q % 2)
        else:
            pltpu.make_async_copy(rbs, acc.at[dv.at[0]], ssb).wait()
    plsc.subcore_barrier()

    def _wb(out_hbm):
        def _ld2(j, buf, sem):
            return pltpu.async_copy(acc.at[pl.ds(s * RPT + j * CH, CH), :],
                                    buf, sem)

        dl2 = {0: _ld2(0, ra, gsa), 1: _ld2(1, rb_, gsb)}
        for j in range(RCH):
            buf, gsem, ssem = ((ra, gsa, ssa) if j % 2 == 0
                               else (rb_, gsb, ssb))
            dl2[j].wait()
            pltpu.async_copy(buf, out_hbm.at[pl.ds(s * RPT + j * CH, CH), :],
                             ssem).wait()
            if j + 2 < RCH:
                dl2[j + 2] = _ld2(j + 2, buf, gsem)

    @pl.when(c == 0)
    def _():
        _wb(out0_hbm)

    @pl.when(c == 1)
    def _():
        _wb(out1_hbm)


def _agg_call(hp, srcp, dstp):
    k = pl.kernel(
        _agg_body,
        out_type=(
            jax.ShapeDtypeStruct((NPAD, FD), jnp.float32),
            jax.ShapeDtypeStruct((NPAD, FD), jnp.float32),
        ),
        mesh=_mesh(),
        scratch_types=[
            pltpu.VMEM((QC, EC), jnp.int32),
            pltpu.VMEM((QC, EC), jnp.int32),
            pltpu.VMEM((QC, EC), jnp.int32),
            pltpu.VMEM((QC, EC), jnp.int32),
            pltpu.VMEM((CH, FD), jnp.float32),
            pltpu.VMEM((CH, FD), jnp.float32),
            pltpu.SemaphoreType.DMA,
            pltpu.SemaphoreType.DMA,
            pltpu.SemaphoreType.DMA,
            pltpu.SemaphoreType.DMA,
            pltpu.SemaphoreType.DMA,
            pltpu.SemaphoreType.DMA,
            pltpu.VMEM_SHARED((NPAD, FD), jnp.float32),
        ],
    )
    return k(hp, srcp, dstp)


BR = 512
TGRID = NPAD // BR


def _dis(d_ref):
    return lax.rsqrt(d_ref[...] + 1.0)


def _scale_body(x_ref, w_ref, d_ref, hp_ref):
    h = jnp.dot(x_ref[...], w_ref[...], preferred_element_type=jnp.float32)
    hp_ref[...] = _dis(d_ref) * h


def _scale_call(x, W1, degs):
    blk = lambda i: (i, 0)
    return pl.pallas_call(
        _scale_body,
        grid=(TGRID,),
        in_specs=[
            pl.BlockSpec((BR, FD), blk),
            pl.BlockSpec((FD, FD), lambda i: (0, 0)),
            pl.BlockSpec((BR, 1), blk),
        ],
        out_specs=pl.BlockSpec((BR, FD), blk),
        out_shape=jax.ShapeDtypeStruct((NPAD, FD), jnp.float32),
    )(x, W1, degs)


def _ln_relu(z, g_ref, be_ref):
    ones_col = jnp.full((FD, 1), 1.0 / FD, jnp.float32)
    mu = jnp.dot(z, ones_col, preferred_element_type=jnp.float32)
    s2 = jnp.dot(z * z, ones_col, preferred_element_type=jnp.float32)
    rs = lax.rsqrt(s2 - mu * mu + EPS)
    g = g_ref[...]
    amat = jnp.dot(rs, g, preferred_element_type=jnp.float32)
    cmat = jnp.dot(mu * rs, g, preferred_element_type=jnp.float32) - be_ref[...]
    return jnp.maximum(z * amat - cmat, 0.0)


def _mid_body(a0_ref, a1_ref, hp_ref, d_ref, b_ref, g_ref, be_ref,
              w2_ref, out_ref):
    dis = _dis(d_ref)
    z = dis * (a0_ref[...] + a1_ref[...] - hp_ref[...]) + b_ref[...]
    r = _ln_relu(z, g_ref, be_ref)
    h2 = jnp.dot(r, w2_ref[...], preferred_element_type=jnp.float32)
    out_ref[...] = dis * h2


def _mid_call(a0, a1, hp, degs, b1, g1, be1, W2):
    blk = lambda i: (i, 0)
    vec = lambda i: (0, 0)
    return pl.pallas_call(
        _mid_body,
        grid=(TGRID,),
        in_specs=[
            pl.BlockSpec((BR, FD), blk),
            pl.BlockSpec((BR, FD), blk),
            pl.BlockSpec((BR, FD), blk),
            pl.BlockSpec((BR, 1), blk),
            pl.BlockSpec((1, FD), vec),
            pl.BlockSpec((1, FD), vec),
            pl.BlockSpec((1, FD), vec),
            pl.BlockSpec((FD, FD), vec),
        ],
        out_specs=pl.BlockSpec((BR, FD), blk),
        out_shape=jax.ShapeDtypeStruct((NPAD, FD), jnp.float32),
    )(a0, a1, hp, degs, b1, g1, be1, W2)


def _final_body(a0_ref, a1_ref, hp_ref, d_ref, b_ref, g_ref, be_ref,
                out_ref):
    i = pl.program_id(0)
    dis = _dis(d_ref)
    z = dis * (a0_ref[...] + a1_ref[...] - hp_ref[...]) + b_ref[...]
    r = _ln_relu(z, g_ref, be_ref)
    rowid = lax.broadcasted_iota(jnp.int32, (BR, FD), 0) + i * BR
    r = jnp.where(rowid < NN, r, 0.0)
    part = jnp.dot(jnp.ones((1, BR), jnp.float32), r,
                   preferred_element_type=jnp.float32)

    @pl.when(i == 0)
    def _():
        out_ref[...] = jnp.zeros((1, FD), jnp.float32)

    out_ref[...] += part

    @pl.when(i == TGRID - 1)
    def _():
        out_ref[...] = out_ref[...] * (1.0 / NN)


def _final_call(a0, a1, hp, degs, b2, g2, be2):
    blk = lambda i: (i, 0)
    vec = lambda i: (0, 0)
    return pl.pallas_call(
        _final_body,
        grid=(TGRID,),
        in_specs=[
            pl.BlockSpec((BR, FD), blk),
            pl.BlockSpec((BR, FD), blk),
            pl.BlockSpec((BR, FD), blk),
            pl.BlockSpec((BR, 1), blk),
            pl.BlockSpec((1, FD), vec),
            pl.BlockSpec((1, FD), vec),
            pl.BlockSpec((1, FD), vec),
        ],
        out_specs=pl.BlockSpec((1, FD), vec),
        out_shape=jax.ShapeDtypeStruct((1, FD), jnp.float32),
    )(a0, a1, hp, degs, b2, g2, be2)


def kernel(x, edge_index, W1, b1, g1, be1, W2, b2, g2, be2):
    srcp = edge_index[0].astype(jnp.int32).reshape(NW * ECW, EC)
    dstp = edge_index[1].astype(jnp.int32).reshape(NW * ECW, EC)

    deg0, deg1 = _deg_call(dstp)
    degs = (deg0 + deg1).reshape(NPAD, 1)
    hp1 = _scale_call(x, W1, degs)
    a10, a11 = _agg_call(hp1, srcp, dstp)
    hp2 = _mid_call(a10, a11, hp1, degs,
                    b1.reshape(1, FD), g1.reshape(1, FD), be1.reshape(1, FD),
                    W2)
    a20, a21 = _agg_call(hp2, srcp, dstp)
    return _final_call(a20, a21, hp2, degs,
                       b2.reshape(1, FD), g2.reshape(1, FD),
                       be2.reshape(1, FD))


# speedup vs baseline: 1.0271x; 1.0016x over previous
import jax
import jax.numpy as jnp
from jax import lax
from jax.experimental import pallas as pl
from jax.experimental.pallas import tpu as pltpu
from jax.experimental.pallas import tpu_sc as plsc

NN = 10000
FD = 128
NE = 320000
NC = 2
NS = 16
NW = NC * NS
NPAD = 10240
RPT = NPAD // NS
CH = 128
RCH = RPT // CH
EC = 125
ECW = NE // NW // EC
QC = 16
NSEG = ECW // QC
EPS = 1e-5


def _mesh():
    return plsc.VectorSubcoreMesh(core_axis_name="c", subcore_axis_name="s")


def _deg_body(dst_hbm, out0_hbm, out1_hbm, dsts_v, ones_v, stg1,
              sem0, sem1, sem2, sem3, acc):
    c = lax.axis_index("c")
    s = lax.axis_index("s")
    w = c * NS + s
    slab = pl.ds(s * RPT, RPT)
    for t in range(128 // 16):
        ones_v[pl.ds(16 * t, 16)] = jnp.ones((16,), jnp.float32)
    for t in range(RPT // 16):
        stg1[pl.ds(16 * t, 16)] = jnp.zeros((16,), jnp.float32)
    pltpu.sync_copy(stg1, acc.at[slab])
    pltpu.sync_copy(dst_hbm.at[pl.ds(w * ECW, ECW), :], dsts_v)
    plsc.subcore_barrier()
    sems = [sem0, sem1, sem2, sem3]

    def body(i, carry):
        descs = []
        for b in range(4):
            descs.append(pltpu.async_copy(
                ones_v.at[pl.ds(0, EC)], acc.at[dsts_v.at[i * 4 + b]],
                sems[b], add=True))
        for d in descs:
            d.wait()
        return carry

    lax.fori_loop(0, ECW // 4, body, 0)
    plsc.subcore_barrier()
    pltpu.sync_copy(acc.at[slab], stg1)

    @pl.when(c == 0)
    def _():
        pltpu.sync_copy(stg1, out0_hbm.at[slab])

    @pl.when(c == 1)
    def _():
        pltpu.sync_copy(stg1, out1_hbm.at[slab])


def _deg_call(dstp):
    k = pl.kernel(
        _deg_body,
        out_type=(
            jax.ShapeDtypeStruct((NPAD,), jnp.float32),
            jax.ShapeDtypeStruct((NPAD,), jnp.float32),
        ),
        mesh=_mesh(),
        scratch_types=[
            pltpu.VMEM((ECW, EC), jnp.int32),
            pltpu.VMEM((128,), jnp.float32),
            pltpu.VMEM((RPT,), jnp.float32),
            pltpu.SemaphoreType.DMA,
            pltpu.SemaphoreType.DMA,
            pltpu.SemaphoreType.DMA,
            pltpu.SemaphoreType.DMA,
            pltpu.VMEM_SHARED((NPAD,), jnp.float32),
        ],
    )
    return k(dstp)


def _agg_body(hp_hbm, src_hbm, dst_hbm, out0_hbm, out1_hbm,
              src0_v, dst0_v, src1_v, dst1_v,
              ra, rb_, gsa, gsb, ssa, ssb, isem0, isem1, acc):
    c = lax.axis_index("c")
    s = lax.axis_index("s")
    w = c * NS + s

    idxsets0 = [(src0_v, dst0_v), (src1_v, dst1_v)]
    isems0 = [isem0, isem1]

    def _refill0(q, p):
        sv, dv = idxsets0[p]
        pltpu.async_copy(src_hbm.at[pl.ds(w * ECW + q * QC, QC), :], sv,
                         isems0[p])
        pltpu.async_copy(dst_hbm.at[pl.ds(w * ECW + q * QC, QC), :], dv,
                         isems0[p])

    _refill0(0, 0)
    _refill0(1, 1)

    def _ld(j, buf, sem):
        return pltpu.async_copy(hp_hbm.at[pl.ds(s * RPT + j * CH, CH), :],
                                buf, sem)

    dl = {0: _ld(0, ra, gsa), 1: _ld(1, rb_, gsb)}
    for j in range(RCH):
        buf, gsem, ssem = (ra, gsa, ssa) if j % 2 == 0 else (rb_, gsb, ssb)
        dl[j].wait()
        pltpu.async_copy(buf, acc.at[pl.ds(s * RPT + j * CH, CH), :],
                         ssem).wait()
        if j + 2 < RCH:
            dl[j + 2] = _ld(j + 2, buf, gsem)
    plsc.subcore_barrier()

    ras = ra.at[pl.ds(0, EC), :]
    rbs = rb_.at[pl.ds(0, EC), :]
    idxsets = [(src0_v, dst0_v), (src1_v, dst1_v)]

    isems = [isem0, isem1]

    def _refill(q, p):
        sv, dv = idxsets[p]
        pltpu.async_copy(src_hbm.at[pl.ds(w * ECW + q * QC, QC), :], sv,
                         isems[p])
        pltpu.async_copy(dst_hbm.at[pl.ds(w * ECW + q * QC, QC), :], dv,
                         isems[p])

    def _refill_wait(p):
        sv, dv = idxsets[p]
        pltpu.make_async_copy(src_hbm.at[pl.ds(w * ECW, QC), :], sv,
                              isems[p]).wait()
        pltpu.make_async_copy(dst_hbm.at[pl.ds(w * ECW, QC), :], dv,
                              isems[p]).wait()

    _refill_wait(0)
    pltpu.async_copy(hp_hbm.at[src0_v.at[0]], ras, gsa)
    for q in range(NSEG):
        sv, dv = idxsets[q % 2]

        def body(i, carry, sv=sv, dv=dv, first=(q == 0)):
            def _wait_prev_b():
                pltpu.make_async_copy(rbs, acc.at[dv.at[0]], ssb).wait()

            if first:
                @pl.when(i > 0)
                def _():
                    _wait_prev_b()
            else:
                _wait_prev_b()
            pltpu.make_async_copy(hp_hbm.at[sv.at[2 * i]], ras, gsa).wait()
            pltpu.async_copy(ras, acc.at[dv.at[2 * i]], ssa, add=True)
            dgb = pltpu.async_copy(hp_hbm.at[sv.at[2 * i + 1]], rbs, gsb)
            dgb.wait()
            pltpu.async_copy(rbs, acc.at[dv.at[2 * i + 1]], ssb, add=True)
            pltpu.make_async_copy(ras, acc.at[dv.at[2 * i]], ssa).wait()

            @pl.when(i < QC // 2 - 1)
            def _():
                pltpu.async_copy(hp_hbm.at[sv.at[2 * i + 2]], ras, gsa)

            return carry

        lax.fori_loop(0, QC // 2, body, 0)
        if q + 1 < NSEG:
            nsv, _ndv = idxsets[(q + 1) % 2]
            _refill_wait((q + 1) % 2)
            pltpu.async_copy(hp_hbm.at[nsv.at[0]], ras, gsa)
            if q + 2 < NSEG:
                _refill(q + 2, q % 2)
        else:
            pltpu.make_async_copy(rbs, acc.at[dv.at[0]], ssb).wait()
    plsc.subcore_barrier()

    def _wb(out_hbm):
        def _ld2(j, buf, sem):
            return pltpu.async_copy(acc.at[pl.ds(s * RPT + j * CH, CH), :],
                                    buf, sem)

        dl2 = {0: _ld2(0, ra, gsa), 1: _ld2(1, rb_, gsb)}
        for j in range(RCH):
            buf, gsem, ssem = ((ra, gsa, ssa) if j % 2 == 0
                               else (rb_, gsb, ssb))
            dl2[j].wait()
            pltpu.async_copy(buf, out_hbm.at[pl.ds(s * RPT + j * CH, CH), :],
                             ssem).wait()
            if j + 2 < RCH:
                dl2[j + 2] = _ld2(j + 2, buf, gsem)

    @pl.when(c == 0)
    def _():
        _wb(out0_hbm)

    @pl.when(c == 1)
    def _():
        _wb(out1_hbm)


def _agg_call(hp, srcp, dstp):
    k = pl.kernel(
        _agg_body,
        out_type=(
            jax.ShapeDtypeStruct((NPAD, FD), jnp.float32),
            jax.ShapeDtypeStruct((NPAD, FD), jnp.float32),
        ),
        mesh=_mesh(),
        scratch_types=[
            pltpu.VMEM((QC, EC), jnp.int32),
            pltpu.VMEM((QC, EC), jnp.int32),
            pltpu.VMEM((QC, EC), jnp.int32),
            pltpu.VMEM((QC, EC), jnp.int32),
            pltpu.VMEM((CH, FD), jnp.float32),
            pltpu.VMEM((CH, FD), jnp.float32),
            pltpu.SemaphoreType.DMA,
            pltpu.SemaphoreType.DMA,
            pltpu.SemaphoreType.DMA,
            pltpu.SemaphoreType.DMA,
            pltpu.SemaphoreType.DMA,
            pltpu.SemaphoreType.DMA,
            pltpu.VMEM_SHARED((NPAD, FD), jnp.float32),
        ],
    )
    return k(hp, srcp, dstp)


BR = 512
TGRID = NPAD // BR


def _dis(d_ref):
    return lax.rsqrt(d_ref[...] + 1.0)


def _scale_body(x_ref, w_ref, d_ref, hp_ref):
    h = jnp.dot(x_ref[...], w_ref[...], preferred_element_type=jnp.float32)
    hp_ref[...] = _dis(d_ref) * h


def _scale_call(x, W1, degs):
    blk = lambda i: (i, 0)
    return pl.pallas_call(
        _scale_body,
        grid=(TGRID,),
        in_specs=[
            pl.BlockSpec((BR, FD), blk),
            pl.BlockSpec((FD, FD), lambda i: (0, 0)),
            pl.BlockSpec((BR, 1), blk),
        ],
        out_specs=pl.BlockSpec((BR, FD), blk),
        out_shape=jax.ShapeDtypeStruct((NPAD, FD), jnp.float32),
    )(x, W1, degs)


def _ln_relu(z, g_ref, be_ref):
    ones_col = jnp.full((FD, 1), 1.0 / FD, jnp.float32)
    mu = jnp.dot(z, ones_col, preferred_element_type=jnp.float32)
    s2 = jnp.dot(z * z, ones_col, preferred_element_type=jnp.float32)
    rs = lax.rsqrt(s2 - mu * mu + EPS)
    g = g_ref[...]
    amat = jnp.dot(rs, g, preferred_element_type=jnp.float32)
    cmat = jnp.dot(mu * rs, g, preferred_element_type=jnp.float32) - be_ref[...]
    return jnp.maximum(z * amat - cmat, 0.0)


def _mid_body(a0_ref, a1_ref, hp_ref, d_ref, b_ref, g_ref, be_ref,
              w2_ref, out_ref):
    dis = _dis(d_ref)
    z = dis * (a0_ref[...] + a1_ref[...] - hp_ref[...]) + b_ref[...]
    r = _ln_relu(z, g_ref, be_ref)
    h2 = jnp.dot(r, w2_ref[...], preferred_element_type=jnp.float32)
    out_ref[...] = dis * h2


def _mid_call(a0, a1, hp, degs, b1, g1, be1, W2):
    blk = lambda i: (i, 0)
    vec = lambda i: (0, 0)
    return pl.pallas_call(
        _mid_body,
        grid=(TGRID,),
        in_specs=[
            pl.BlockSpec((BR, FD), blk),
            pl.BlockSpec((BR, FD), blk),
            pl.BlockSpec((BR, FD), blk),
            pl.BlockSpec((BR, 1), blk),
            pl.BlockSpec((1, FD), vec),
            pl.BlockSpec((1, FD), vec),
            pl.BlockSpec((1, FD), vec),
            pl.BlockSpec((FD, FD), vec),
        ],
        out_specs=pl.BlockSpec((BR, FD), blk),
        out_shape=jax.ShapeDtypeStruct((NPAD, FD), jnp.float32),
    )(a0, a1, hp, degs, b1, g1, be1, W2)


def _final_body(a0_ref, a1_ref, hp_ref, d_ref, b_ref, g_ref, be_ref,
                out_ref):
    i = pl.program_id(0)
    dis = _dis(d_ref)
    z = dis * (a0_ref[...] + a1_ref[...] - hp_ref[...]) + b_ref[...]
    r = _ln_relu(z, g_ref, be_ref)
    rowid = lax.broadcasted_iota(jnp.int32, (BR, FD), 0) + i * BR
    r = jnp.where(rowid < NN, r, 0.0)
    part = jnp.dot(jnp.ones((1, BR), jnp.float32), r,
                   preferred_element_type=jnp.float32)

    @pl.when(i == 0)
    def _():
        out_ref[...] = jnp.zeros((1, FD), jnp.float32)

    out_ref[...] += part

    @pl.when(i == TGRID - 1)
    def _():
        out_ref[...] = out_ref[...] * (1.0 / NN)


def _final_call(a0, a1, hp, degs, b2, g2, be2):
    blk = lambda i: (i, 0)
    vec = lambda i: (0, 0)
    return pl.pallas_call(
        _final_body,
        grid=(TGRID,),
        in_specs=[
            pl.BlockSpec((BR, FD), blk),
            pl.BlockSpec((BR, FD), blk),
            pl.BlockSpec((BR, FD), blk),
            pl.BlockSpec((BR, 1), blk),
            pl.BlockSpec((1, FD), vec),
            pl.BlockSpec((1, FD), vec),
            pl.BlockSpec((1, FD), vec),
        ],
        out_specs=pl.BlockSpec((1, FD), vec),
        out_shape=jax.ShapeDtypeStruct((1, FD), jnp.float32),
    )(a0, a1, hp, degs, b2, g2, be2)


def kernel(x, edge_index, W1, b1, g1, be1, W2, b2, g2, be2):
    srcp = edge_index[0].astype(jnp.int32).reshape(NW * ECW, EC)
    dstp = edge_index[1].astype(jnp.int32).reshape(NW * ECW, EC)

    deg0, deg1 = _deg_call(dstp)
    degs = (deg0 + deg1).reshape(NPAD, 1)
    hp1 = _scale_call(x, W1, degs)
    a10, a11 = _agg_call(hp1, srcp, dstp)
    hp2 = _mid_call(a10, a11, hp1, degs,
                    b1.reshape(1, FD), g1.reshape(1, FD), be1.reshape(1, FD),
                    W2)
    a20, a21 = _agg_call(hp2, srcp, dstp)
    return _final_call(a20, a21, hp2, degs,
                       b2.reshape(1, FD), g2.reshape(1, FD),
                       be2.reshape(1, FD))
